# baseline (speedup 1.0000x reference)
import jax
import jax.numpy as jnp
from jax import lax
from jax.experimental import pallas as pl
from jax.experimental.pallas import tpu as pltpu
from jax.experimental.pallas import tpu_sc as plsc

KMER_NUM = 100000
CLASS_NUM = 100
DIM = 32
B = 1024
L = 200
LP = 256

NC = 2
NS = 16
NW = NC * NS
B_PER_W = B // NW
IDX_COLS = 128
IDX_ROWS = B * LP // IDX_COLS
IDX_ROWS_W = IDX_ROWS // NW
DMAS_PER_READ = LP // IDX_COLS
HALF = 16
PK = DIM // 2
ROWS_PER_TILE = KMER_NUM // NS
PC = 125
NPC = ROWS_PER_TILE // PC


def _sc_body(embs_hbm, idx_hbm, out_hbm, tab_sh, idx_v,
             fbuf0, fbuf1, pbuf0, pbuf1, gbuf0, gbuf1, acc_v,
             psem0, psem1, gsem0, gsem1):
    cid = lax.axis_index("c")
    sid = lax.axis_index("s")
    wid = sid * NC + cid

    pltpu.sync_copy(idx_hbm.at[pl.ds(wid * IDX_ROWS_W, IDX_ROWS_W)], idx_v)

    fbufs, pbufs, psems = (fbuf0, fbuf1), (pbuf0, pbuf1), (psem0, psem1)
    t0 = sid * ROWS_PER_TILE
    c7fff = jnp.full((HALF,), 0x7FFF, jnp.int32)
    c1 = jnp.full((HALF,), 1, jnp.int32)
    chi = jnp.full((HALF,), -65536, jnp.int32)

    def rtne(u):
        odd = lax.bitwise_and(lax.shift_right_logical(u, 16), c1)
        return u + c7fff + odd

    PACK_UNROLL = 5

    def pack_chunk(k, slot):
        fbuf, pbuf = fbufs[slot], pbufs[slot]

        def body(m, carry):
            i0 = m * PACK_UNROLL
            for d in range(PACK_UNROLL):
                i = i0 + d
                u1 = plsc.bitcast(fbuf[i, 0:HALF], jnp.int32)
                u2 = plsc.bitcast(fbuf[i, HALF:DIM], jnp.int32)
                lo = lax.shift_right_logical(rtne(u1), 16)
                hi = lax.bitwise_and(rtne(u2), chi)
                pbuf[i, 0:PK] = lax.bitwise_or(lo, hi)
            return carry

        lax.fori_loop(0, PC // PACK_UNROLL, body, 0)
        pltpu.sync_copy(pbuf, tab_sh.at[pl.ds(t0 + k * PC, PC)])

    inflight = pltpu.async_copy(embs_hbm.at[pl.ds(t0, PC)], fbufs[0], psems[0])
    for k in range(NPC):
        slot = k % 2
        cur = inflight
        if k + 1 < NPC:
            inflight = pltpu.async_copy(
                embs_hbm.at[pl.ds(t0 + (k + 1) * PC, PC)],
                fbufs[(k + 1) % 2], psems[(k + 1) % 2])
        cur.wait()
        pack_chunk(k, slot)

    plsc.subcore_barrier()

    gbufs, gsems = (gbuf0, gbuf1), (gsem0, gsem1)

    def fire(r, slot):
        handles = []
        for j in range(DMAS_PER_READ):
            h = pltpu.async_copy(
                tab_sh.at[idx_v.at[r * DMAS_PER_READ + j]],
                gbufs[slot].at[pl.ds(j * IDX_COLS, IDX_COLS)],
                gsems[slot])
            handles.append(h)
        return handles

    inflight = fire(0, 0)
    for r in range(B_PER_W):
        slot = r % 2
        cur = inflight
        if r + 1 < B_PER_W:
            inflight = fire(r + 1, (r + 1) % 2)
        for h in cur:
            h.wait()
        gbuf = gbufs[slot]
        ACC_UNROLL = 4

        def body(m, carry, gbuf=gbuf):
            accs = list(carry)
            l0 = m * ACC_UNROLL
            for d in range(ACC_UNROLL):
                v = gbuf[l0 + d, 0:PK]
                lo = plsc.bitcast(lax.shift_left(v, 16), jnp.float32)
                hi = plsc.bitcast(lax.bitwise_and(v, chi), jnp.float32)
                accs[2 * d] = accs[2 * d] + lo
                accs[2 * d + 1] = accs[2 * d + 1] + hi
            return tuple(accs)

        z = jnp.zeros((HALF,), jnp.float32)
        accs = lax.fori_loop(0, L // ACC_UNROLL, body, (z,) * (2 * ACC_UNROLL))
        acc_v[r, 0:HALF] = (accs[0] + accs[2]) + (accs[4] + accs[6])
        acc_v[r, HALF:DIM] = (accs[1] + accs[3]) + (accs[5] + accs[7])

    pltpu.sync_copy(acc_v, out_hbm.at[pl.ds(wid * B_PER_W, B_PER_W)])


def _gather_sum(embs, idx_mat):
    mesh = plsc.VectorSubcoreMesh(core_axis_name="c", subcore_axis_name="s")
    fn = pl.kernel(
        _sc_body,
        out_type=jax.ShapeDtypeStruct((B, DIM), jnp.float32),
        mesh=mesh,
        scratch_types=[
            pltpu.VMEM_SHARED((KMER_NUM, PK), jnp.int32),
            pltpu.VMEM((IDX_ROWS_W, IDX_COLS), jnp.int32),
            pltpu.VMEM((PC, DIM), jnp.float32),
            pltpu.VMEM((PC, DIM), jnp.float32),
            pltpu.VMEM((PC, PK), jnp.int32),
            pltpu.VMEM((PC, PK), jnp.int32),
            pltpu.VMEM((LP, PK), jnp.int32),
            pltpu.VMEM((LP, PK), jnp.int32),
            pltpu.VMEM((B_PER_W, DIM), jnp.float32),
            pltpu.SemaphoreType.DMA,
            pltpu.SemaphoreType.DMA,
            pltpu.SemaphoreType.DMA,
            pltpu.SemaphoreType.DMA,
        ],
        compiler_params=pltpu.CompilerParams(use_tc_tiling_on_sc=False,
                                             needs_layout_passes=False),
    )
    return fn(embs, idx_mat)


def _loss_body(emb_ref, w_ref, lab_ref, out_ref):
    logits = lax.dot_general(
        emb_ref[...], w_ref[...],
        dimension_numbers=(((1,), (1,)), ((), ())),
        preferred_element_type=jnp.float32)
    m = jnp.max(logits, axis=1, keepdims=True)
    lse = m + jnp.log(jnp.sum(jnp.exp(logits - m), axis=1, keepdims=True))
    cls = lax.broadcasted_iota(jnp.int32, logits.shape, 1)
    picked = jnp.sum(jnp.where(cls == lab_ref[...], logits, 0.0),
                     axis=1, keepdims=True)
    out_ref[...] = jnp.sum(lse - picked, keepdims=True)


def _loss(read_emb, softmax_weights, read_labels):
    out = pl.pallas_call(
        _loss_body,
        out_shape=jax.ShapeDtypeStruct((1, 1), jnp.float32),
    )(read_emb, softmax_weights, read_labels.reshape(B, 1))
    return out[0, 0]


@jax.jit
def kernel(reads, read_labels, embs, softmax_weights):
    reads_p = jnp.pad(reads, ((0, 0), (0, LP - L)))
    idx_mat = reads_p.reshape(IDX_ROWS, IDX_COLS)
    read_emb = _gather_sum(embs, idx_mat)
    return _loss(read_emb, softmax_weights, read_labels)


# speedup vs baseline: 1.0026x; 1.0026x over previous
import jax
import jax.numpy as jnp
from jax import lax
from jax.experimental import pallas as pl
from jax.experimental.pallas import tpu as pltpu
from jax.experimental.pallas import tpu_sc as plsc

KMER_NUM = 100000
CLASS_NUM = 100
DIM = 32
B = 1024
L = 200
LP = 256

NC = 2
NS = 16
NW = NC * NS
B_PER_W = B // NW
IDX_COLS = 128
IDX_ROWS = B * LP // IDX_COLS
IDX_ROWS_W = IDX_ROWS // NW
DMAS_PER_READ = LP // IDX_COLS
HALF = 16
PK = DIM // 2
ROWS_PER_TILE = KMER_NUM // NS
PC = 125
NPC = ROWS_PER_TILE // PC


def _sc_body(embs_hbm, idx_hbm, out_hbm, tab_sh, idx_v,
             fbuf0, fbuf1, pbuf0, pbuf1, gbuf0, gbuf1, acc_v,
             psem0, psem1, gsem0, gsem1):
    cid = lax.axis_index("c")
    sid = lax.axis_index("s")
    wid = sid * NC + cid

    pltpu.sync_copy(idx_hbm.at[pl.ds(wid * IDX_ROWS_W, IDX_ROWS_W)], idx_v)

    fbufs, pbufs, psems = (fbuf0, fbuf1), (pbuf0, pbuf1), (psem0, psem1)
    t0 = sid * ROWS_PER_TILE
    c7fff = jnp.full((HALF,), 0x7FFF, jnp.int32)
    c1 = jnp.full((HALF,), 1, jnp.int32)
    chi = jnp.full((HALF,), -65536, jnp.int32)

    def rtne(u):
        odd = lax.bitwise_and(lax.shift_right_logical(u, 16), c1)
        return u + c7fff + odd

    PACK_UNROLL = 5

    def pack_chunk(k, slot):
        fbuf, pbuf = fbufs[slot], pbufs[slot]

        def body(m, carry):
            i0 = m * PACK_UNROLL
            for d in range(PACK_UNROLL):
                i = i0 + d
                u1 = plsc.bitcast(fbuf[i, 0:HALF], jnp.int32)
                u2 = plsc.bitcast(fbuf[i, HALF:DIM], jnp.int32)
                lo = lax.shift_right_logical(rtne(u1), 16)
                hi = lax.bitwise_and(rtne(u2), chi)
                pbuf[i, 0:PK] = lax.bitwise_or(lo, hi)
            return carry

        lax.fori_loop(0, PC // PACK_UNROLL, body, 0)
        pltpu.sync_copy(pbuf, tab_sh.at[pl.ds(t0 + k * PC, PC)])

    with jax.named_scope("pack_phase"):
        inflight = pltpu.async_copy(
            embs_hbm.at[pl.ds(t0, PC)], fbufs[0], psems[0])
        for k in range(NPC):
            slot = k % 2
            cur = inflight
            if k + 1 < NPC:
                inflight = pltpu.async_copy(
                    embs_hbm.at[pl.ds(t0 + (k + 1) * PC, PC)],
                    fbufs[(k + 1) % 2], psems[(k + 1) % 2])
            cur.wait()
            pack_chunk(k, slot)

        plsc.subcore_barrier()

    gbufs, gsems = (gbuf0, gbuf1), (gsem0, gsem1)
    gather_scope = jax.named_scope("gather_phase")
    gather_scope.__enter__()

    def fire(r, slot):
        handles = []
        for j in range(DMAS_PER_READ):
            h = pltpu.async_copy(
                tab_sh.at[idx_v.at[r * DMAS_PER_READ + j]],
                gbufs[slot].at[pl.ds(j * IDX_COLS, IDX_COLS)],
                gsems[slot])
            handles.append(h)
        return handles

    inflight = fire(0, 0)
    for r in range(B_PER_W):
        slot = r % 2
        cur = inflight
        if r + 1 < B_PER_W:
            inflight = fire(r + 1, (r + 1) % 2)
        for h in cur:
            h.wait()
        gbuf = gbufs[slot]
        ACC_UNROLL = 4

        def body(m, carry, gbuf=gbuf):
            accs = list(carry)
            l0 = m * ACC_UNROLL
            for d in range(ACC_UNROLL):
                v = gbuf[l0 + d, 0:PK]
                lo = plsc.bitcast(lax.shift_left(v, 16), jnp.float32)
                hi = plsc.bitcast(lax.bitwise_and(v, chi), jnp.float32)
                accs[2 * d] = accs[2 * d] + lo
                accs[2 * d + 1] = accs[2 * d + 1] + hi
            return tuple(accs)

        z = jnp.zeros((HALF,), jnp.float32)
        accs = lax.fori_loop(0, L // ACC_UNROLL, body, (z,) * (2 * ACC_UNROLL))
        acc_v[r, 0:HALF] = (accs[0] + accs[2]) + (accs[4] + accs[6])
        acc_v[r, HALF:DIM] = (accs[1] + accs[3]) + (accs[5] + accs[7])

    pltpu.sync_copy(acc_v, out_hbm.at[pl.ds(wid * B_PER_W, B_PER_W)])
    gather_scope.__exit__(None, None, None)


def _gather_sum(embs, idx_mat):
    mesh = plsc.VectorSubcoreMesh(core_axis_name="c", subcore_axis_name="s")
    fn = pl.kernel(
        _sc_body,
        out_type=jax.ShapeDtypeStruct((B, DIM), jnp.float32),
        mesh=mesh,
        scratch_types=[
            pltpu.VMEM_SHARED((KMER_NUM, PK), jnp.int32),
            pltpu.VMEM((IDX_ROWS_W, IDX_COLS), jnp.int32),
            pltpu.VMEM((PC, DIM), jnp.float32),
            pltpu.VMEM((PC, DIM), jnp.float32),
            pltpu.VMEM((PC, PK), jnp.int32),
            pltpu.VMEM((PC, PK), jnp.int32),
            pltpu.VMEM((LP, PK), jnp.int32),
            pltpu.VMEM((LP, PK), jnp.int32),
            pltpu.VMEM((B_PER_W, DIM), jnp.float32),
            pltpu.SemaphoreType.DMA,
            pltpu.SemaphoreType.DMA,
            pltpu.SemaphoreType.DMA,
            pltpu.SemaphoreType.DMA,
        ],
        compiler_params=pltpu.CompilerParams(use_tc_tiling_on_sc=False,
                                             needs_layout_passes=False),
    )
    return fn(embs, idx_mat)


def _loss_body(emb_ref, w_ref, lab_ref, out_ref):
    logits = lax.dot_general(
        emb_ref[...], w_ref[...],
        dimension_numbers=(((1,), (1,)), ((), ())),
        preferred_element_type=jnp.float32)
    m = jnp.max(logits, axis=1, keepdims=True)
    lse = m + jnp.log(jnp.sum(jnp.exp(logits - m), axis=1, keepdims=True))
    cls = lax.broadcasted_iota(jnp.int32, logits.shape, 1)
    picked = jnp.sum(jnp.where(cls == lab_ref[...], logits, 0.0),
                     axis=1, keepdims=True)
    out_ref[...] = jnp.sum(lse - picked, keepdims=True)


def _loss(read_emb, softmax_weights, read_labels):
    out = pl.pallas_call(
        _loss_body,
        out_shape=jax.ShapeDtypeStruct((1, 1), jnp.float32),
    )(read_emb, softmax_weights, read_labels.reshape(B, 1))
    return out[0, 0]


@jax.jit
def kernel(reads, read_labels, embs, softmax_weights):
    reads_p = jnp.pad(reads, ((0, 0), (0, LP - L)))
    idx_mat = reads_p.reshape(IDX_ROWS, IDX_COLS)
    read_emb = _gather_sum(embs, idx_mat)
    return _loss(read_emb, softmax_weights, read_labels)


# speedup vs baseline: 1.1261x; 1.1231x over previous
import jax
import jax.numpy as jnp
from jax import lax
from jax.experimental import pallas as pl
from jax.experimental.pallas import tpu as pltpu
from jax.experimental.pallas import tpu_sc as plsc

KMER_NUM = 100000
CLASS_NUM = 100
DIM = 32
B = 1024
L = 200
LP = 256

NC = 2
NS = 16
NW = NC * NS
B_PER_W = B // NW
IDX_COLS = 128
IDX_ROWS = B * LP // IDX_COLS
IDX_ROWS_W = IDX_ROWS // NW
DMAS_PER_READ = LP // IDX_COLS
HALF = 16
PK = DIM // 2
ROWS_PER_TILE = KMER_NUM // NS
PC = 125
NPC = ROWS_PER_TILE // PC


def _sc_body(embs_hbm, idx_hbm, out_hbm, tab_sh, idx_v,
             fbuf0, fbuf1, pbuf0, pbuf1, gbuf0, gbuf1, acc_v,
             psem0, psem1, gsem0, gsem1, osem0, osem1):
    cid = lax.axis_index("c")
    sid = lax.axis_index("s")
    wid = sid * NC + cid

    pltpu.sync_copy(idx_hbm.at[pl.ds(wid * IDX_ROWS_W, IDX_ROWS_W)], idx_v)

    fbufs, pbufs, psems = (fbuf0, fbuf1), (pbuf0, pbuf1), (psem0, psem1)
    osems = (osem0, osem1)
    t0 = sid * ROWS_PER_TILE
    c8000 = jnp.full((HALF,), 0x8000, jnp.int32)
    chi = jnp.full((HALF,), -65536, jnp.int32)

    PACK_UNROLL = 5

    def pack_chunk(slot):
        fbuf, pbuf = fbufs[slot], pbufs[slot]

        def body(m, carry):
            i0 = m * PACK_UNROLL
            for d in range(PACK_UNROLL):
                i = i0 + d
                u1 = plsc.bitcast(fbuf[i, 0:HALF], jnp.int32)
                u2 = plsc.bitcast(fbuf[i, HALF:DIM], jnp.int32)
                lo = lax.shift_right_logical(u1 + c8000, 16)
                hi = lax.bitwise_and(u2 + c8000, chi)
                pbuf[i, 0:PK] = lax.bitwise_or(lo, hi)
            return carry

        lax.fori_loop(0, PC // PACK_UNROLL, body, 0)

    with jax.named_scope("pack_phase"):
        inflight = pltpu.async_copy(
            embs_hbm.at[pl.ds(t0, PC)], fbufs[0], psems[0])
        out_inflight = [None, None]
        for k in range(NPC):
            slot = k % 2
            cur = inflight
            if k + 1 < NPC:
                inflight = pltpu.async_copy(
                    embs_hbm.at[pl.ds(t0 + (k + 1) * PC, PC)],
                    fbufs[(k + 1) % 2], psems[(k + 1) % 2])
            cur.wait()
            if out_inflight[slot] is not None:
                out_inflight[slot].wait()
            pack_chunk(slot)
            out_inflight[slot] = pltpu.async_copy(
                pbufs[slot], tab_sh.at[pl.ds(t0 + k * PC, PC)], osems[slot])
        for h in out_inflight:
            h.wait()

        plsc.subcore_barrier()

    gbufs, gsems = (gbuf0, gbuf1), (gsem0, gsem1)
    gather_scope = jax.named_scope("gather_phase")
    gather_scope.__enter__()

    def fire(r, slot):
        handles = []
        for j in range(DMAS_PER_READ):
            h = pltpu.async_copy(
                tab_sh.at[idx_v.at[r * DMAS_PER_READ + j]],
                gbufs[slot].at[pl.ds(j * IDX_COLS, IDX_COLS)],
                gsems[slot])
            handles.append(h)
        return handles

    inflight = fire(0, 0)
    for r in range(B_PER_W):
        slot = r % 2
        cur = inflight
        if r + 1 < B_PER_W:
            inflight = fire(r + 1, (r + 1) % 2)
        for h in cur:
            h.wait()
        gbuf = gbufs[slot]
        ACC_UNROLL = 4

        def body(m, carry, gbuf=gbuf):
            accs = list(carry)
            l0 = m * ACC_UNROLL
            for d in range(ACC_UNROLL):
                v = gbuf[l0 + d, 0:PK]
                lo = plsc.bitcast(lax.shift_left(v, 16), jnp.float32)
                hi = plsc.bitcast(lax.bitwise_and(v, chi), jnp.float32)
                accs[2 * d] = accs[2 * d] + lo
                accs[2 * d + 1] = accs[2 * d + 1] + hi
            return tuple(accs)

        z = jnp.zeros((HALF,), jnp.float32)
        accs = lax.fori_loop(0, L // ACC_UNROLL, body, (z,) * (2 * ACC_UNROLL))
        acc_v[r, 0:HALF] = (accs[0] + accs[2]) + (accs[4] + accs[6])
        acc_v[r, HALF:DIM] = (accs[1] + accs[3]) + (accs[5] + accs[7])

    pltpu.sync_copy(acc_v, out_hbm.at[pl.ds(wid * B_PER_W, B_PER_W)])
    gather_scope.__exit__(None, None, None)


def _gather_sum(embs, idx_mat):
    mesh = plsc.VectorSubcoreMesh(core_axis_name="c", subcore_axis_name="s")
    fn = pl.kernel(
        _sc_body,
        out_type=jax.ShapeDtypeStruct((B, DIM), jnp.float32),
        mesh=mesh,
        scratch_types=[
            pltpu.VMEM_SHARED((KMER_NUM, PK), jnp.int32),
            pltpu.VMEM((IDX_ROWS_W, IDX_COLS), jnp.int32),
            pltpu.VMEM((PC, DIM), jnp.float32),
            pltpu.VMEM((PC, DIM), jnp.float32),
            pltpu.VMEM((PC, PK), jnp.int32),
            pltpu.VMEM((PC, PK), jnp.int32),
            pltpu.VMEM((LP, PK), jnp.int32),
            pltpu.VMEM((LP, PK), jnp.int32),
            pltpu.VMEM((B_PER_W, DIM), jnp.float32),
            pltpu.SemaphoreType.DMA,
            pltpu.SemaphoreType.DMA,
            pltpu.SemaphoreType.DMA,
            pltpu.SemaphoreType.DMA,
            pltpu.SemaphoreType.DMA,
            pltpu.SemaphoreType.DMA,
        ],
        compiler_params=pltpu.CompilerParams(use_tc_tiling_on_sc=False,
                                             needs_layout_passes=False),
    )
    return fn(embs, idx_mat)


def _loss_body(emb_ref, w_ref, lab_ref, out_ref):
    logits = lax.dot_general(
        emb_ref[...], w_ref[...],
        dimension_numbers=(((1,), (1,)), ((), ())),
        preferred_element_type=jnp.float32)
    m = jnp.max(logits, axis=1, keepdims=True)
    lse = m + jnp.log(jnp.sum(jnp.exp(logits - m), axis=1, keepdims=True))
    cls = lax.broadcasted_iota(jnp.int32, logits.shape, 1)
    picked = jnp.sum(jnp.where(cls == lab_ref[...], logits, 0.0),
                     axis=1, keepdims=True)
    out_ref[...] = jnp.sum(lse - picked, keepdims=True)


def _loss(read_emb, softmax_weights, read_labels):
    out = pl.pallas_call(
        _loss_body,
        out_shape=jax.ShapeDtypeStruct((1, 1), jnp.float32),
    )(read_emb, softmax_weights, read_labels.reshape(B, 1))
    return out[0, 0]


@jax.jit
def kernel(reads, read_labels, embs, softmax_weights):
    reads_p = jnp.pad(reads, ((0, 0), (0, LP - L)))
    idx_mat = reads_p.reshape(IDX_ROWS, IDX_COLS)
    read_emb = _gather_sum(embs, idx_mat)
    return _loss(read_emb, softmax_weights, read_labels)


# speedup vs baseline: 1.2490x; 1.1092x over previous
import jax
import jax.numpy as jnp
from jax import lax
from jax.experimental import pallas as pl
from jax.experimental.pallas import tpu as pltpu
from jax.experimental.pallas import tpu_sc as plsc

KMER_NUM = 100000
CLASS_NUM = 100
DIM = 32
B = 1024
L = 200

NC = 2
NS = 16
NW = NC * NS
B_PER_W = B // NW
IDX_COLS = 128
HALF = 16
PK = DIM // 2
ROWS_PER_TILE = KMER_NUM // NS
PC = 125
NPC = ROWS_PER_TILE // PC


def _sc_body(embs_hbm, idx_hbm, out_hbm, tab_sh, idx_v,
             fbuf0, fbuf1, pbuf0, pbuf1, gbuf0, gbuf1, acc_v,
             psem0, psem1, gsem0, gsem1, osem0, osem1):
    cid = lax.axis_index("c")
    sid = lax.axis_index("s")
    wid = sid * NC + cid

    pltpu.sync_copy(idx_hbm.at[pl.ds(wid * B_PER_W, B_PER_W)], idx_v)

    fbufs, pbufs, psems = (fbuf0, fbuf1), (pbuf0, pbuf1), (psem0, psem1)
    osems = (osem0, osem1)
    t0 = sid * ROWS_PER_TILE
    c8000 = jnp.full((HALF,), 0x8000, jnp.int32)
    chi = jnp.full((HALF,), -65536, jnp.int32)

    PACK_UNROLL = 5

    def pack_chunk(slot):
        fbuf, pbuf = fbufs[slot], pbufs[slot]

        def body(m, carry):
            i0 = m * PACK_UNROLL
            for d in range(PACK_UNROLL):
                i = i0 + d
                u1 = plsc.bitcast(fbuf[i, 0:HALF], jnp.int32)
                u2 = plsc.bitcast(fbuf[i, HALF:DIM], jnp.int32)
                lo = lax.shift_right_logical(u1 + c8000, 16)
                hi = lax.bitwise_and(u2 + c8000, chi)
                pbuf[i, 0:PK] = lax.bitwise_or(lo, hi)
            return carry

        lax.fori_loop(0, PC // PACK_UNROLL, body, 0)

    with jax.named_scope("pack_phase"):
        inflight = pltpu.async_copy(
            embs_hbm.at[pl.ds(t0, PC)], fbufs[0], psems[0])
        out_inflight = [None, None]
        for k in range(NPC):
            slot = k % 2
            cur = inflight
            if k + 1 < NPC:
                inflight = pltpu.async_copy(
                    embs_hbm.at[pl.ds(t0 + (k + 1) * PC, PC)],
                    fbufs[(k + 1) % 2], psems[(k + 1) % 2])
            cur.wait()
            if out_inflight[slot] is not None:
                out_inflight[slot].wait()
            pack_chunk(slot)
            out_inflight[slot] = pltpu.async_copy(
                pbufs[slot], tab_sh.at[pl.ds(t0 + k * PC, PC)], osems[slot])
        for h in out_inflight:
            h.wait()

        plsc.subcore_barrier()

    gbufs, gsems = (gbuf0, gbuf1), (gsem0, gsem1)
    gather_scope = jax.named_scope("gather_phase")
    gather_scope.__enter__()

    def fire(r, slot):
        h0 = pltpu.async_copy(
            tab_sh.at[idx_v.at[r, pl.ds(0, IDX_COLS)]],
            gbufs[slot].at[pl.ds(0, IDX_COLS)], gsems[slot])
        h1 = pltpu.async_copy(
            tab_sh.at[idx_v.at[r, pl.ds(IDX_COLS, L - IDX_COLS)]],
            gbufs[slot].at[pl.ds(IDX_COLS, L - IDX_COLS)], gsems[slot])
        return [h0, h1]

    inflight = fire(0, 0)
    for r in range(B_PER_W):
        slot = r % 2
        cur = inflight
        if r + 1 < B_PER_W:
            inflight = fire(r + 1, (r + 1) % 2)
        for h in cur:
            h.wait()
        gbuf = gbufs[slot]
        ACC_UNROLL = 4

        def body(m, carry, gbuf=gbuf):
            accs = list(carry)
            l0 = m * ACC_UNROLL
            for d in range(ACC_UNROLL):
                v = gbuf[l0 + d, 0:PK]
                lo = plsc.bitcast(lax.shift_left(v, 16), jnp.float32)
                hi = plsc.bitcast(lax.bitwise_and(v, chi), jnp.float32)
                accs[2 * d] = accs[2 * d] + lo
                accs[2 * d + 1] = accs[2 * d + 1] + hi
            return tuple(accs)

        z = jnp.zeros((HALF,), jnp.float32)
        accs = lax.fori_loop(0, L // ACC_UNROLL, body, (z,) * (2 * ACC_UNROLL))
        acc_v[r, 0:HALF] = (accs[0] + accs[2]) + (accs[4] + accs[6])
        acc_v[r, HALF:DIM] = (accs[1] + accs[3]) + (accs[5] + accs[7])

    pltpu.sync_copy(acc_v, out_hbm.at[pl.ds(wid * B_PER_W, B_PER_W)])
    gather_scope.__exit__(None, None, None)


def _gather_sum(embs, idx_mat):
    mesh = plsc.VectorSubcoreMesh(core_axis_name="c", subcore_axis_name="s")
    fn = pl.kernel(
        _sc_body,
        out_type=jax.ShapeDtypeStruct((B, DIM), jnp.float32),
        mesh=mesh,
        scratch_types=[
            pltpu.VMEM_SHARED((KMER_NUM, PK), jnp.int32),
            pltpu.VMEM((B_PER_W, L), jnp.int32),
            pltpu.VMEM((PC, DIM), jnp.float32),
            pltpu.VMEM((PC, DIM), jnp.float32),
            pltpu.VMEM((PC, PK), jnp.int32),
            pltpu.VMEM((PC, PK), jnp.int32),
            pltpu.VMEM((L, PK), jnp.int32),
            pltpu.VMEM((L, PK), jnp.int32),
            pltpu.VMEM((B_PER_W, DIM), jnp.float32),
            pltpu.SemaphoreType.DMA,
            pltpu.SemaphoreType.DMA,
            pltpu.SemaphoreType.DMA,
            pltpu.SemaphoreType.DMA,
            pltpu.SemaphoreType.DMA,
            pltpu.SemaphoreType.DMA,
        ],
        compiler_params=pltpu.CompilerParams(use_tc_tiling_on_sc=False,
                                             needs_layout_passes=False),
    )
    return fn(embs, idx_mat)


def _loss_body(emb_ref, w_ref, lab_ref, out_ref):
    logits = lax.dot_general(
        emb_ref[...], w_ref[...],
        dimension_numbers=(((1,), (1,)), ((), ())),
        preferred_element_type=jnp.float32)
    m = jnp.max(logits, axis=1, keepdims=True)
    lse = m + jnp.log(jnp.sum(jnp.exp(logits - m), axis=1, keepdims=True))
    cls = lax.broadcasted_iota(jnp.int32, logits.shape, 1)
    picked = jnp.sum(jnp.where(cls == lab_ref[...], logits, 0.0),
                     axis=1, keepdims=True)
    out_ref[...] = jnp.sum(lse - picked, keepdims=True)


def _loss(read_emb, softmax_weights, read_labels):
    out = pl.pallas_call(
        _loss_body,
        out_shape=jax.ShapeDtypeStruct((1, 1), jnp.float32),
    )(read_emb, softmax_weights, read_labels.reshape(B, 1))
    return out[0, 0]


@jax.jit
def kernel(reads, read_labels, embs, softmax_weights):
    read_emb = _gather_sum(embs, reads)
    return _loss(read_emb, softmax_weights, read_labels)


# speedup vs baseline: 1.2506x; 1.0013x over previous
import jax
import jax.numpy as jnp
from jax import lax
from jax.experimental import pallas as pl
from jax.experimental.pallas import tpu as pltpu
from jax.experimental.pallas import tpu_sc as plsc

KMER_NUM = 100000
CLASS_NUM = 100
DIM = 32
B = 1024
L = 200

NC = 2
NS = 16
NW = NC * NS
B_PER_W = B // NW
IDX_COLS = 128
HALF = 16
PK = DIM // 2
ROWS_PER_TILE = KMER_NUM // NS
PC = 125
NPC = ROWS_PER_TILE // PC


def _sc_body(embs_hbm, idx_hbm, out_hbm, tab_sh, idx_v,
             fbuf0, fbuf1, pbuf0, pbuf1, gbuf0, gbuf1, acc_v,
             psem0, psem1, gsem0, gsem1, osem0, osem1):
    cid = lax.axis_index("c")
    sid = lax.axis_index("s")
    wid = sid * NC + cid

    pltpu.sync_copy(idx_hbm.at[pl.ds(wid * B_PER_W * L, B_PER_W * L)], idx_v)

    fbufs, pbufs, psems = (fbuf0, fbuf1), (pbuf0, pbuf1), (psem0, psem1)
    osems = (osem0, osem1)
    t0 = sid * ROWS_PER_TILE
    f0 = t0 * DIM
    c8000 = jnp.full((HALF,), 0x8000, jnp.int32)
    chi = jnp.full((HALF,), -65536, jnp.int32)

    PACK_UNROLL = 5

    def pack_chunk(slot):
        fbuf, pbuf = fbufs[slot], pbufs[slot]

        def body(m, carry):
            i0 = m * PACK_UNROLL
            for d in range(PACK_UNROLL):
                i = i0 + d
                u1 = plsc.bitcast(fbuf[pl.ds(i * DIM, HALF)], jnp.int32)
                u2 = plsc.bitcast(fbuf[pl.ds(i * DIM + HALF, HALF)], jnp.int32)
                lo = lax.shift_right_logical(u1 + c8000, 16)
                hi = lax.bitwise_and(u2 + c8000, chi)
                pbuf[i, 0:PK] = lax.bitwise_or(lo, hi)
            return carry

        lax.fori_loop(0, PC // PACK_UNROLL, body, 0)

    with jax.named_scope("pack_phase"):
        inflight = pltpu.async_copy(
            embs_hbm.at[pl.ds(f0, PC * DIM)], fbufs[0], psems[0])
        out_inflight = [None, None]
        for k in range(NPC):
            slot = k % 2
            cur = inflight
            if k + 1 < NPC:
                inflight = pltpu.async_copy(
                    embs_hbm.at[pl.ds(f0 + (k + 1) * PC * DIM, PC * DIM)],
                    fbufs[(k + 1) % 2], psems[(k + 1) % 2])
            cur.wait()
            if out_inflight[slot] is not None:
                out_inflight[slot].wait()
            pack_chunk(slot)
            out_inflight[slot] = pltpu.async_copy(
                pbufs[slot], tab_sh.at[pl.ds(t0 + k * PC, PC)], osems[slot])
        for h in out_inflight:
            h.wait()

        plsc.subcore_barrier()

    gbufs, gsems = (gbuf0, gbuf1), (gsem0, gsem1)
    gather_scope = jax.named_scope("gather_phase")
    gather_scope.__enter__()

    def fire(r, slot):
        h0 = pltpu.async_copy(
            tab_sh.at[idx_v.at[pl.ds(r * L, IDX_COLS)]],
            gbufs[slot].at[pl.ds(0, IDX_COLS)], gsems[slot])
        h1 = pltpu.async_copy(
            tab_sh.at[idx_v.at[pl.ds(r * L + IDX_COLS, L - IDX_COLS)]],
            gbufs[slot].at[pl.ds(IDX_COLS, L - IDX_COLS)], gsems[slot])
        return [h0, h1]

    inflight = fire(0, 0)
    for r in range(B_PER_W):
        slot = r % 2
        cur = inflight
        if r + 1 < B_PER_W:
            inflight = fire(r + 1, (r + 1) % 2)
        for h in cur:
            h.wait()
        gbuf = gbufs[slot]
        ACC_UNROLL = 4

        def body(m, carry, gbuf=gbuf):
            accs = list(carry)
            l0 = m * ACC_UNROLL
            for d in range(ACC_UNROLL):
                v = gbuf[l0 + d, 0:PK]
                lo = plsc.bitcast(lax.shift_left(v, 16), jnp.float32)
                hi = plsc.bitcast(lax.bitwise_and(v, chi), jnp.float32)
                accs[2 * d] = accs[2 * d] + lo
                accs[2 * d + 1] = accs[2 * d + 1] + hi
            return tuple(accs)

        z = jnp.zeros((HALF,), jnp.float32)
        accs = lax.fori_loop(0, L // ACC_UNROLL, body, (z,) * (2 * ACC_UNROLL))
        acc_v[pl.ds(r * DIM, HALF)] = (accs[0] + accs[2]) + (accs[4] + accs[6])
        acc_v[pl.ds(r * DIM + HALF, HALF)] = (
            (accs[1] + accs[3]) + (accs[5] + accs[7]))

    pltpu.sync_copy(acc_v, out_hbm.at[pl.ds(wid * B_PER_W * DIM,
                                            B_PER_W * DIM)])
    gather_scope.__exit__(None, None, None)


def _gather_sum(embs_flat, reads_flat):
    mesh = plsc.VectorSubcoreMesh(core_axis_name="c", subcore_axis_name="s")
    fn = pl.kernel(
        _sc_body,
        out_type=jax.ShapeDtypeStruct((B * DIM,), jnp.float32),
        mesh=mesh,
        scratch_types=[
            pltpu.VMEM_SHARED((KMER_NUM, PK), jnp.int32),
            pltpu.VMEM((B_PER_W * L,), jnp.int32),
            pltpu.VMEM((PC * DIM,), jnp.float32),
            pltpu.VMEM((PC * DIM,), jnp.float32),
            pltpu.VMEM((PC, PK), jnp.int32),
            pltpu.VMEM((PC, PK), jnp.int32),
            pltpu.VMEM((L, PK), jnp.int32),
            pltpu.VMEM((L, PK), jnp.int32),
            pltpu.VMEM((B_PER_W * DIM,), jnp.float32),
            pltpu.SemaphoreType.DMA,
            pltpu.SemaphoreType.DMA,
            pltpu.SemaphoreType.DMA,
            pltpu.SemaphoreType.DMA,
            pltpu.SemaphoreType.DMA,
            pltpu.SemaphoreType.DMA,
        ],
        compiler_params=pltpu.CompilerParams(use_tc_tiling_on_sc=False,
                                             needs_layout_passes=False),
    )
    return fn(embs_flat, reads_flat)


def _loss_body(emb_ref, w_ref, lab_ref, out_ref):
    logits = lax.dot_general(
        emb_ref[...], w_ref[...],
        dimension_numbers=(((1,), (1,)), ((), ())),
        preferred_element_type=jnp.float32)
    m = jnp.max(logits, axis=1, keepdims=True)
    lse = m + jnp.log(jnp.sum(jnp.exp(logits - m), axis=1, keepdims=True))
    cls = lax.broadcasted_iota(jnp.int32, logits.shape, 1)
    picked = jnp.sum(jnp.where(cls == lab_ref[...], logits, 0.0),
                     axis=1, keepdims=True)
    out_ref[...] = jnp.sum(lse - picked, keepdims=True)


def _loss(read_emb, softmax_weights, read_labels):
    out = pl.pallas_call(
        _loss_body,
        out_shape=jax.ShapeDtypeStruct((1, 1), jnp.float32),
    )(read_emb, softmax_weights, read_labels.reshape(B, 1))
    return out[0, 0]


@jax.jit
def kernel(reads, read_labels, embs, softmax_weights):
    read_emb_flat = _gather_sum(embs.reshape(-1), reads.reshape(-1))
    read_emb = read_emb_flat.reshape(B, DIM)
    return _loss(read_emb, softmax_weights, read_labels)
